# baseline (speedup 1.0000x reference)
import functools

import jax
import jax.numpy as jnp
from jax import lax
from jax.experimental import pallas as pl
from jax.experimental.pallas import tpu as pltpu
from jax.experimental.pallas import tpu_sc as plsc

_B, _K, _D = 4096, 8192, 32
_DP = 128
_BB = 512
_NW = 32
_BPW = _B // _NW


def _sc_body(cb_hbm, codes_hbm, hard_hbm, idx_v, rows_v, sem):
    wid = lax.axis_index("s") * 2 + lax.axis_index("c")
    base = wid * _BPW
    pltpu.sync_copy(codes_hbm.at[pl.ds(base, _BPW)], idx_v)
    pltpu.async_copy(cb_hbm.at[idx_v], rows_v, sem).wait()
    pltpu.sync_copy(rows_v, hard_hbm.at[pl.ds(base, _BPW)])


_sc_gather = pl.kernel(
    _sc_body,
    out_type=jax.ShapeDtypeStruct((_B, _DP), jnp.float32),
    mesh=plsc.VectorSubcoreMesh(core_axis_name="c", subcore_axis_name="s"),
    scratch_types=[
        pltpu.VMEM((_BPW,), jnp.int32),
        pltpu.VMEM((_BPW, _DP), jnp.float32),
        pltpu.SemaphoreType.DMA,
    ],
)


_KH = _K // 2


def _tc_body(xl_ref, xr_ref, codes_ref, cb_ref, hard_ref, out_ref):
    xl = xl_ref[...]
    xr = xr_ref[...]
    m = jnp.maximum(jnp.max(xl, axis=1, keepdims=True),
                    jnp.max(xr, axis=1, keepdims=True))
    el = jnp.exp(xl - m)
    er = jnp.exp(xr - m)
    t = (jnp.sum(el * xl, axis=1, keepdims=True)
         + jnp.sum(er * xr, axis=1, keepdims=True))

    codes = codes_ref[...]
    iota = lax.broadcasted_iota(jnp.int32, (_BB, _KH), 1)
    ohl = iota == codes
    ohr = (iota + _KH) == codes
    l_code = (jnp.sum(jnp.where(ohl, xl, 0.0), axis=1, keepdims=True)
              + jnp.sum(jnp.where(ohr, xr, 0.0), axis=1, keepdims=True))

    dn = (((1,), (0,)), ((), ()))
    cb = cb_ref[...]
    ones_h = jnp.ones((_KH, 1), jnp.bfloat16)
    cbl = jnp.concatenate([cb[:_KH], ones_h], axis=1)
    cbr = jnp.concatenate([cb[_KH:], ones_h], axis=1)
    vs = (lax.dot_general(el.astype(jnp.bfloat16), cbl, dn,
                          preferred_element_type=jnp.float32)
          + lax.dot_general(er.astype(jnp.bfloat16), cbr, dn,
                            preferred_element_type=jnp.float32))
    v = vs[:, :_D]
    s = vs[:, _D:]
    logs = jnp.log(s)

    out_ref[...] = jnp.concatenate(
        [hard_ref[:, :_D], v / s, l_code - m - logs, m + logs - t / s],
        axis=1)


@jax.jit
def kernel(logits, codes, codebook):
    cb_pad = jnp.pad(codebook, ((0, 0), (0, _DP - _D)))
    hard = _sc_gather(cb_pad, codes)
    return pl.pallas_call(
        _tc_body,
        grid=(_B // _BB,),
        in_specs=[
            pl.BlockSpec((_BB, _KH), lambda i: (i, 0)),
            pl.BlockSpec((_BB, _KH), lambda i: (i, 1)),
            pl.BlockSpec((_BB, 1), lambda i: (i, 0)),
            pl.BlockSpec((_K, _D), lambda i: (0, 0)),
            pl.BlockSpec((_BB, _DP), lambda i: (i, 0)),
        ],
        out_specs=pl.BlockSpec((_BB, 2 * _D + 2), lambda i: (i, 0)),
        out_shape=jax.ShapeDtypeStruct((_B, 2 * _D + 2), jnp.float32),
        compiler_params=pltpu.CompilerParams(vmem_limit_bytes=120 * 1024 * 1024),
    )(logits, logits, codes.reshape(_B, 1), codebook.astype(jnp.bfloat16),
      hard)


# speedup vs baseline: 1.0979x; 1.0979x over previous
import functools

import jax
import jax.numpy as jnp
from jax import lax
from jax.experimental import pallas as pl
from jax.experimental.pallas import tpu as pltpu
from jax.experimental.pallas import tpu_sc as plsc

_B, _K, _D = 4096, 8192, 32
_DP = 128
_BB = 512
_NW = 32
_BPW = _B // _NW


def _sc_body(cb_hbm, codes_hbm, hard_hbm, idx_v, rows_v, sem):
    wid = lax.axis_index("s") * 2 + lax.axis_index("c")
    base = wid * _BPW
    pltpu.sync_copy(codes_hbm.at[pl.ds(base, _BPW)], idx_v)
    pltpu.async_copy(cb_hbm.at[idx_v], rows_v, sem).wait()
    pltpu.sync_copy(rows_v, hard_hbm.at[pl.ds(base, _BPW)])


_sc_gather = pl.kernel(
    _sc_body,
    out_type=jax.ShapeDtypeStruct((_B, _DP), jnp.float32),
    mesh=plsc.VectorSubcoreMesh(core_axis_name="c", subcore_axis_name="s"),
    scratch_types=[
        pltpu.VMEM((_BPW,), jnp.int32),
        pltpu.VMEM((_BPW, _DP), jnp.float32),
        pltpu.SemaphoreType.DMA,
    ],
)


def _tc_body(logits_ref, codes_ref, cb_ref, hard_ref, out_ref):
    x = logits_ref[...]
    m = jnp.max(x, axis=1, keepdims=True)
    e = jnp.exp(x - m)
    t = jnp.sum(e * x, axis=1, keepdims=True)

    codes = codes_ref[...]
    iota = lax.broadcasted_iota(jnp.int32, (_BB, _K), 1)
    oh = iota == codes
    l_code = jnp.sum(jnp.where(oh, x, 0.0), axis=1, keepdims=True)

    dn = (((1,), (0,)), ((), ()))
    eb = e.astype(jnp.bfloat16)
    cbe = jnp.concatenate(
        [cb_ref[...], jnp.ones((_K, 1), jnp.bfloat16)], axis=1)
    vs = lax.dot_general(eb, cbe, dn,
                         preferred_element_type=jnp.float32)
    v = vs[:, :_D]
    s = vs[:, _D:]
    logs = jnp.log(s)

    out_ref[...] = jnp.concatenate(
        [hard_ref[:, :_D], v / s, l_code - m - logs, m + logs - t / s],
        axis=1)


@jax.jit
def kernel(logits, codes, codebook):
    cb_pad = jnp.pad(codebook, ((0, 0), (0, _DP - _D)))
    hard = _sc_gather(cb_pad, codes)
    return pl.pallas_call(
        _tc_body,
        grid=(_B // _BB,),
        in_specs=[
            pl.BlockSpec((_BB, _K), lambda i: (i, 0)),
            pl.BlockSpec((_BB, 1), lambda i: (i, 0)),
            pl.BlockSpec((_K, _D), lambda i: (0, 0)),
            pl.BlockSpec((_BB, _DP), lambda i: (i, 0)),
        ],
        out_specs=pl.BlockSpec((_BB, 2 * _D + 2), lambda i: (i, 0)),
        out_shape=jax.ShapeDtypeStruct((_B, 2 * _D + 2), jnp.float32),
        compiler_params=pltpu.CompilerParams(vmem_limit_bytes=120 * 1024 * 1024),
    )(logits, codes.reshape(_B, 1), codebook.astype(jnp.bfloat16), hard)
